# baseline (speedup 1.0000x reference)
import functools

import jax
import jax.numpy as jnp
from jax import lax
from jax.experimental import pallas as pl
from jax.experimental.pallas import tpu as pltpu
from jax.experimental.pallas import tpu_sc as plsc

N = 10000
E = 320000
D_IN, D_HID, D_OUT = 128, 64, 16

NC, NS = 2, 16
NW = NC * NS
KB = 128
EPAD = 7680
EP = E + EPAD
KN = EP // (NW * KB)
NBUF = 5
H = 2
NG = KN // NBUF
DFL = 20
NDUMP = 8
ROWCH = 200
NCH = 1000 // ROWCH


def _mesh():
    return plsc.VectorSubcoreMesh(
        core_axis_name="c", subcore_axis_name="s",
        num_cores=NC, num_subcores=NS)


_SC_PARAMS = pltpu.CompilerParams(use_tc_tiling_on_sc=False)


def _deg_body(dst_hbm, z1_hbm, out_hbm, acc, dst_v, ones_v, obuf, dsem):
    cid = lax.axis_index("c")
    sid = lax.axis_index("s")
    wid = sid * NC + cid

    @pl.when(sid < 10)
    def _init():
        pltpu.sync_copy(z1_hbm.at[pl.ds(sid * 1000, 1000)], obuf)
        pltpu.sync_copy(obuf, acc.at[pl.ds(sid * 1000, 1000)])

    for i in range(KB // 16):
        ones_v[pl.ds(i * 16, 16)] = jnp.ones((16,), jnp.float32)
    plsc.subcore_barrier()

    pltpu.sync_copy(dst_hbm.at[wid], dst_v)
    ones = ones_v

    def group(g, c):
        for i in range(DFL):
            pltpu.async_copy(ones, acc.at[dst_v.at[g * DFL + i]],
                             dsem, add=True)
        for i in range(DFL):
            pltpu.make_async_copy(
                ones, acc.at[dst_v.at[g * DFL + i]], dsem).wait()
        return c
    lax.fori_loop(0, KN // DFL, group, 0)
    plsc.subcore_barrier()

    @pl.when(sid < 10)
    def _out():
        pltpu.sync_copy(acc.at[pl.ds(sid * 1000, 1000)], obuf)
        pltpu.sync_copy(obuf, out_hbm.at[pl.ds(cid * N + sid * 1000, 1000)])


_deg = functools.partial(
    pl.kernel,
    out_type=jax.ShapeDtypeStruct((NC * N,), jnp.float32),
    mesh=_mesh(),
    compiler_params=_SC_PARAMS,
    scratch_types=[
        pltpu.VMEM_SHARED((N + NDUMP,), jnp.float32),
        pltpu.VMEM((KN, KB), jnp.int32),
        pltpu.VMEM((KB,), jnp.float32),
        pltpu.VMEM((1000,), jnp.float32),
        pltpu.SemaphoreType.DMA,
    ])(_deg_body)


def _agg_body(h_hbm, z_hbm, src_hbm, dst_hbm, out_hbm,
              acc, src_v, dst_v, obuf, *ring):
    rows = ring[:NBUF]
    gsem = ring[NBUF:2 * NBUF]
    ssem = ring[2 * NBUF:3 * NBUF]
    cid = lax.axis_index("c")
    sid = lax.axis_index("s")
    wid = sid * NC + cid

    @pl.when(sid < 10)
    def _init():
        for k in range(NCH):
            r0 = sid * 1000 + k * ROWCH
            pltpu.sync_copy(z_hbm.at[pl.ds(r0, ROWCH)], obuf)
            pltpu.sync_copy(obuf, acc.at[pl.ds(r0, ROWCH)])
    plsc.subcore_barrier()

    pltpu.sync_copy(src_hbm.at[wid], src_v)
    pltpu.sync_copy(dst_hbm.at[wid], dst_v)

    for b in range(H):
        pltpu.async_copy(h_hbm.at[src_v.at[b]], rows[b], gsem[b])

    def group(g, c):
        for b in range(NBUF):
            j = g * NBUF + b
            pltpu.make_async_copy(
                h_hbm.at[src_v.at[j]], rows[b], gsem[b]).wait()
            pltpu.async_copy(rows[b], acc.at[dst_v.at[j]], ssem[b], add=True)
            h = (b + H) % NBUF
            jg = j + H

            @pl.when(jg < KN)
            def _prefetch(h=h, jg=jg):
                @pl.when(jg >= NBUF)
                def _wait_scatter():
                    pltpu.make_async_copy(
                        rows[h], acc.at[dst_v.at[jg - NBUF]], ssem[h]).wait()
                pltpu.async_copy(h_hbm.at[src_v.at[jg]], rows[h], gsem[h])
        return c
    lax.fori_loop(0, NG, group, 0)
    for b in range(NBUF):
        pltpu.make_async_copy(
            rows[b], acc.at[dst_v.at[KN - NBUF + b]], ssem[b]).wait()
    plsc.subcore_barrier()

    @pl.when(sid < 10)
    def _out():
        for k in range(NCH):
            r0 = sid * 1000 + k * ROWCH
            pltpu.sync_copy(acc.at[pl.ds(r0, ROWCH)], obuf)
            pltpu.sync_copy(obuf, out_hbm.at[cid, pl.ds(r0, ROWCH)])


def _make_agg(d):
    return functools.partial(
        pl.kernel,
        out_type=jax.ShapeDtypeStruct((NC, N, d), jnp.float32),
        mesh=_mesh(),
        compiler_params=_SC_PARAMS,
        scratch_types=(
            [pltpu.VMEM_SHARED((N + NDUMP, d), jnp.float32),
             pltpu.VMEM((KN, KB), jnp.int32),
             pltpu.VMEM((KN, KB), jnp.int32),
             pltpu.VMEM((ROWCH, d), jnp.float32)]
            + [pltpu.VMEM((KB, d), jnp.float32) for _ in range(NBUF)]
            + [pltpu.SemaphoreType.DMA for _ in range(2 * NBUF)]
        ))(_agg_body)


_agg64 = _make_agg(D_HID)
_agg16 = _make_agg(D_OUT)


BR = 1000


def _tc1_body(x_ref, w_ref, deg_ref, h_ref, dinv_ref):
    dinv = lax.rsqrt(deg_ref[...] + 1.0)
    h = jnp.dot(x_ref[...], w_ref[...], preferred_element_type=jnp.float32)
    h_ref[...] = h * dinv
    dinv_ref[...] = dinv


_tc1 = pl.pallas_call(
    _tc1_body,
    grid=(N // BR,),
    in_specs=[pl.BlockSpec((BR, D_IN), lambda i: (i, 0)),
              pl.BlockSpec((D_IN, D_HID), lambda i: (0, 0)),
              pl.BlockSpec((BR, 1), lambda i: (i, 0))],
    out_specs=[pl.BlockSpec((BR, D_HID), lambda i: (i, 0)),
               pl.BlockSpec((BR, 1), lambda i: (i, 0))],
    out_shape=[jax.ShapeDtypeStruct((N, D_HID), jnp.float32),
               jax.ShapeDtypeStruct((N, 1), jnp.float32)])


def _tc_mid_body(p_ref, hp_ref, dinv_ref, b_ref, w_ref, out_ref):
    t = p_ref[0] + p_ref[1] + hp_ref[...]
    t = t * dinv_ref[...] + b_ref[...]
    t = jnp.maximum(t, 0.0)
    out_ref[...] = jnp.dot(
        t, w_ref[...], preferred_element_type=jnp.float32) * dinv_ref[...]


_tc_mid = pl.pallas_call(
    _tc_mid_body,
    grid=(N // BR,),
    in_specs=[pl.BlockSpec((NC, BR, D_HID), lambda i: (0, i, 0)),
              pl.BlockSpec((BR, D_HID), lambda i: (i, 0)),
              pl.BlockSpec((BR, 1), lambda i: (i, 0)),
              pl.BlockSpec((1, D_HID), lambda i: (0, 0)),
              pl.BlockSpec((D_HID, D_OUT), lambda i: (0, 0))],
    out_specs=pl.BlockSpec((BR, D_OUT), lambda i: (i, 0)),
    out_shape=jax.ShapeDtypeStruct((N, D_OUT), jnp.float32))


def _tc_out_body(p_ref, hp_ref, dinv_ref, b_ref, out_ref):
    t = (p_ref[0] + p_ref[1] + hp_ref[...]) * dinv_ref[...] + b_ref[...]
    m = jnp.max(t, axis=1, keepdims=True)
    e = jnp.exp(t - m)
    s = jnp.sum(e, axis=1, keepdims=True)
    out_ref[...] = (t - m) - jnp.log(s)


_tc_out = pl.pallas_call(
    _tc_out_body,
    grid=(N // BR,),
    in_specs=[pl.BlockSpec((NC, BR, D_OUT), lambda i: (0, i, 0)),
              pl.BlockSpec((BR, D_OUT), lambda i: (i, 0)),
              pl.BlockSpec((BR, 1), lambda i: (i, 0)),
              pl.BlockSpec((1, D_OUT), lambda i: (0, 0))],
    out_specs=pl.BlockSpec((BR, D_OUT), lambda i: (i, 0)),
    out_shape=jax.ShapeDtypeStruct((N, D_OUT), jnp.float32))


def kernel(x, edge_index, W1, b1, W2, b2):
    ei = edge_index.astype(jnp.int32)
    pad = jnp.arange(EPAD, dtype=jnp.int32)
    src3 = jnp.concatenate([ei[0], (pad * 8) % N]).reshape(NW, KN, KB)
    dst3 = jnp.concatenate([ei[1], N + (pad % NDUMP)]).reshape(NW, KN, KB)
    z1 = jnp.zeros((N,), jnp.float32)
    z64 = jnp.zeros((N, D_HID), jnp.float32)
    z16 = jnp.zeros((N, D_OUT), jnp.float32)

    degp = _deg(dst3, z1).reshape(NC, N)
    deg = (degp[0] + degp[1]).reshape(N, 1)
    h1p, dinv = _tc1(x, W1, deg)
    p1 = _agg64(h1p, z64, src3, dst3)
    h2p = _tc_mid(p1, h1p, dinv, b1.reshape(1, D_HID), W2)
    p2 = _agg16(h2p, z16, src3, dst3)
    return _tc_out(p2, h2p, dinv, b2.reshape(1, D_OUT))


# speedup vs baseline: 1.1207x; 1.1207x over previous
import functools

import jax
import jax.numpy as jnp
from jax import lax
from jax.experimental import pallas as pl
from jax.experimental.pallas import tpu as pltpu
from jax.experimental.pallas import tpu_sc as plsc

N = 10000
E = 320000
D_IN, D_HID, D_OUT = 128, 64, 16

NC, NS = 2, 16
NW = NC * NS
KB = 128
EPAD = 7680
EP = E + EPAD
KN = EP // (NW * KB)
NBUF = 8
H = 4
NG = KN // NBUF
DFL = 20
NDUMP = 8
RPT = N // NS
IOCH = (KB, KB, KB, KB, RPT - 4 * KB)


def _mesh():
    return plsc.VectorSubcoreMesh(
        core_axis_name="c", subcore_axis_name="s",
        num_cores=NC, num_subcores=NS)


_SC_PARAMS = pltpu.CompilerParams(use_tc_tiling_on_sc=False)


def _deg_body(dst_hbm, z1_hbm, out_hbm, acc, dst_v, ones_v, obuf, dsem):
    cid = lax.axis_index("c")
    sid = lax.axis_index("s")
    wid = sid * NC + cid

    @pl.when(sid < 10)
    def _init():
        pltpu.sync_copy(z1_hbm.at[pl.ds(sid * 1000, 1000)], obuf)
        pltpu.sync_copy(obuf, acc.at[pl.ds(sid * 1000, 1000)])

    for i in range(KB // 16):
        ones_v[pl.ds(i * 16, 16)] = jnp.ones((16,), jnp.float32)
    plsc.subcore_barrier()

    pltpu.sync_copy(dst_hbm.at[wid], dst_v)
    ones = ones_v

    def group(g, c):
        for i in range(DFL):
            pltpu.async_copy(ones, acc.at[dst_v.at[g * DFL + i]],
                             dsem, add=True)
        for i in range(DFL):
            pltpu.make_async_copy(
                ones, acc.at[dst_v.at[g * DFL + i]], dsem).wait()
        return c
    lax.fori_loop(0, KN // DFL, group, 0)
    plsc.subcore_barrier()

    @pl.when(sid < 10)
    def _out():
        pltpu.sync_copy(acc.at[pl.ds(sid * 1000, 1000)], obuf)
        pltpu.sync_copy(obuf, out_hbm.at[pl.ds(cid * N + sid * 1000, 1000)])


_deg = functools.partial(
    pl.kernel,
    out_type=jax.ShapeDtypeStruct((NC * N,), jnp.float32),
    mesh=_mesh(),
    compiler_params=_SC_PARAMS,
    scratch_types=[
        pltpu.VMEM_SHARED((N + NDUMP,), jnp.float32),
        pltpu.VMEM((KN, KB), jnp.int32),
        pltpu.VMEM((KB,), jnp.float32),
        pltpu.VMEM((1000,), jnp.float32),
        pltpu.SemaphoreType.DMA,
    ])(_deg_body)


def _agg_body(h_hbm, z_hbm, src_hbm, dst_hbm, out_hbm,
              acc, src_v, dst_v, *ring):
    rows = ring[:NBUF]
    gsem = ring[NBUF:2 * NBUF]
    ssem = ring[2 * NBUF:3 * NBUF]
    cid = lax.axis_index("c")
    sid = lax.axis_index("s")
    wid = sid * NC + cid

    r0 = sid * RPT
    for k, ch in enumerate(IOCH):
        pltpu.sync_copy(z_hbm.at[pl.ds(r0, ch)], rows[k].at[pl.ds(0, ch)])
        pltpu.sync_copy(rows[k].at[pl.ds(0, ch)], acc.at[pl.ds(r0, ch)])
        r0 = r0 + ch
    plsc.subcore_barrier()

    pltpu.sync_copy(src_hbm.at[wid], src_v)
    pltpu.sync_copy(dst_hbm.at[wid], dst_v)

    for b in range(H):
        pltpu.async_copy(h_hbm.at[src_v.at[b]], rows[b], gsem[b])

    def group(g, c):
        for b in range(NBUF):
            j = g * NBUF + b
            pltpu.make_async_copy(
                h_hbm.at[src_v.at[j]], rows[b], gsem[b]).wait()
            pltpu.async_copy(rows[b], acc.at[dst_v.at[j]], ssem[b], add=True)
            h = (b + H) % NBUF
            jg = j + H

            @pl.when(jg < KN)
            def _prefetch(h=h, jg=jg):
                @pl.when(jg >= NBUF)
                def _wait_scatter():
                    pltpu.make_async_copy(
                        rows[h], acc.at[dst_v.at[jg - NBUF]], ssem[h]).wait()
                pltpu.async_copy(h_hbm.at[src_v.at[jg]], rows[h], gsem[h])
        return c
    lax.fori_loop(0, NG, group, 0)
    for b in range(NBUF):
        pltpu.make_async_copy(
            rows[b], acc.at[dst_v.at[KN - NBUF + b]], ssem[b]).wait()
    plsc.subcore_barrier()

    r1 = sid * RPT
    for k, ch in enumerate(IOCH):
        pltpu.sync_copy(acc.at[pl.ds(r1, ch)], rows[k].at[pl.ds(0, ch)])
        pltpu.sync_copy(rows[k].at[pl.ds(0, ch)], out_hbm.at[cid, pl.ds(r1, ch)])
        r1 = r1 + ch


def _make_agg(d):
    return functools.partial(
        pl.kernel,
        out_type=jax.ShapeDtypeStruct((NC, N, d), jnp.float32),
        mesh=_mesh(),
        compiler_params=_SC_PARAMS,
        scratch_types=(
            [pltpu.VMEM_SHARED((N + NDUMP, d), jnp.float32),
             pltpu.VMEM((KN, KB), jnp.int32),
             pltpu.VMEM((KN, KB), jnp.int32)]
            + [pltpu.VMEM((KB, d), jnp.float32) for _ in range(NBUF)]
            + [pltpu.SemaphoreType.DMA for _ in range(2 * NBUF)]
        ))(_agg_body)


_agg64 = _make_agg(D_HID)
_agg16 = _make_agg(D_OUT)


BR = 1000


def _tc1_body(x_ref, w_ref, deg_ref, h_ref, dinv_ref):
    dinv = lax.rsqrt(deg_ref[...] + 1.0)
    h = jnp.dot(x_ref[...], w_ref[...], preferred_element_type=jnp.float32)
    h_ref[...] = h * dinv
    dinv_ref[...] = dinv


_tc1 = pl.pallas_call(
    _tc1_body,
    grid=(N // BR,),
    in_specs=[pl.BlockSpec((BR, D_IN), lambda i: (i, 0)),
              pl.BlockSpec((D_IN, D_HID), lambda i: (0, 0)),
              pl.BlockSpec((BR, 1), lambda i: (i, 0))],
    out_specs=[pl.BlockSpec((BR, D_HID), lambda i: (i, 0)),
               pl.BlockSpec((BR, 1), lambda i: (i, 0))],
    out_shape=[jax.ShapeDtypeStruct((N, D_HID), jnp.float32),
               jax.ShapeDtypeStruct((N, 1), jnp.float32)])


def _tc_mid_body(p_ref, hp_ref, dinv_ref, b_ref, w_ref, out_ref):
    t = p_ref[0] + p_ref[1] + hp_ref[...]
    t = t * dinv_ref[...] + b_ref[...]
    t = jnp.maximum(t, 0.0)
    out_ref[...] = jnp.dot(
        t, w_ref[...], preferred_element_type=jnp.float32) * dinv_ref[...]


_tc_mid = pl.pallas_call(
    _tc_mid_body,
    grid=(N // BR,),
    in_specs=[pl.BlockSpec((NC, BR, D_HID), lambda i: (0, i, 0)),
              pl.BlockSpec((BR, D_HID), lambda i: (i, 0)),
              pl.BlockSpec((BR, 1), lambda i: (i, 0)),
              pl.BlockSpec((1, D_HID), lambda i: (0, 0)),
              pl.BlockSpec((D_HID, D_OUT), lambda i: (0, 0))],
    out_specs=pl.BlockSpec((BR, D_OUT), lambda i: (i, 0)),
    out_shape=jax.ShapeDtypeStruct((N, D_OUT), jnp.float32))


def _tc_out_body(p_ref, hp_ref, dinv_ref, b_ref, out_ref):
    t = (p_ref[0] + p_ref[1] + hp_ref[...]) * dinv_ref[...] + b_ref[...]
    m = jnp.max(t, axis=1, keepdims=True)
    e = jnp.exp(t - m)
    s = jnp.sum(e, axis=1, keepdims=True)
    out_ref[...] = (t - m) - jnp.log(s)


_tc_out = pl.pallas_call(
    _tc_out_body,
    grid=(N // BR,),
    in_specs=[pl.BlockSpec((NC, BR, D_OUT), lambda i: (0, i, 0)),
              pl.BlockSpec((BR, D_OUT), lambda i: (i, 0)),
              pl.BlockSpec((BR, 1), lambda i: (i, 0)),
              pl.BlockSpec((1, D_OUT), lambda i: (0, 0))],
    out_specs=pl.BlockSpec((BR, D_OUT), lambda i: (i, 0)),
    out_shape=jax.ShapeDtypeStruct((N, D_OUT), jnp.float32))


def kernel(x, edge_index, W1, b1, W2, b2):
    ei = edge_index.astype(jnp.int32)
    pad = jnp.arange(EPAD, dtype=jnp.int32)
    src3 = jnp.concatenate([ei[0], (pad * 8) % N]).reshape(NW, KN, KB)
    dst3 = jnp.concatenate([ei[1], N + (pad % NDUMP)]).reshape(NW, KN, KB)
    z1 = jnp.zeros((N,), jnp.float32)
    z64 = jnp.zeros((N, D_HID), jnp.float32)
    z16 = jnp.zeros((N, D_OUT), jnp.float32)

    degp = _deg(dst3, z1).reshape(NC, N)
    deg = (degp[0] + degp[1]).reshape(N, 1)
    h1p, dinv = _tc1(x, W1, deg)
    p1 = _agg64(h1p, z64, src3, dst3)
    h2p = _tc_mid(p1, h1p, dinv, b1.reshape(1, D_HID), W2)
    p2 = _agg16(h2p, z16, src3, dst3)
    return _tc_out(p2, h2p, dinv, b2.reshape(1, D_OUT))


# speedup vs baseline: 1.2138x; 1.0831x over previous
import functools

import jax
import jax.numpy as jnp
from jax import lax
from jax.experimental import pallas as pl
from jax.experimental.pallas import tpu as pltpu
from jax.experimental.pallas import tpu_sc as plsc

N = 10000
E = 320000
D_IN, D_HID, D_OUT = 128, 64, 16

NC, NS = 2, 16
NW = NC * NS
KB = 128
EPAD = 7680
EP = E + EPAD
KN = EP // (NW * KB)
NBUF = 8
H = 6
NG = KN // NBUF
DFL = 20
NDUMP = 8
RPT = N // NS
IOCH = (KB, KB, KB, KB, RPT - 4 * KB)


def _mesh():
    return plsc.VectorSubcoreMesh(
        core_axis_name="c", subcore_axis_name="s",
        num_cores=NC, num_subcores=NS)


_SC_PARAMS = pltpu.CompilerParams(use_tc_tiling_on_sc=False)


def _deg_body(dst_hbm, z1_hbm, out_hbm, acc, dst_v, ones_v, obuf, dsem):
    cid = lax.axis_index("c")
    sid = lax.axis_index("s")
    wid = sid * NC + cid

    @pl.when(sid < 10)
    def _init():
        pltpu.sync_copy(z1_hbm.at[pl.ds(sid * 1000, 1000)], obuf)
        pltpu.sync_copy(obuf, acc.at[pl.ds(sid * 1000, 1000)])

    for i in range(KB // 16):
        ones_v[pl.ds(i * 16, 16)] = jnp.ones((16,), jnp.float32)
    plsc.subcore_barrier()

    pltpu.sync_copy(dst_hbm.at[wid], dst_v)
    ones = ones_v

    def group(g, c):
        for i in range(DFL):
            pltpu.async_copy(ones, acc.at[dst_v.at[g * DFL + i]],
                             dsem, add=True)
        for i in range(DFL):
            pltpu.make_async_copy(
                ones, acc.at[dst_v.at[g * DFL + i]], dsem).wait()
        return c
    lax.fori_loop(0, KN // DFL, group, 0)
    plsc.subcore_barrier()

    @pl.when(sid < 10)
    def _out():
        pltpu.sync_copy(acc.at[pl.ds(sid * 1000, 1000)], obuf)
        pltpu.sync_copy(obuf, out_hbm.at[pl.ds(cid * N + sid * 1000, 1000)])


_deg = functools.partial(
    pl.kernel,
    out_type=jax.ShapeDtypeStruct((NC * N,), jnp.float32),
    mesh=_mesh(),
    compiler_params=_SC_PARAMS,
    scratch_types=[
        pltpu.VMEM_SHARED((N + NDUMP,), jnp.float32),
        pltpu.VMEM((KN, KB), jnp.int32),
        pltpu.VMEM((KB,), jnp.float32),
        pltpu.VMEM((1000,), jnp.float32),
        pltpu.SemaphoreType.DMA,
    ])(_deg_body)


def _agg_body(h_hbm, z_hbm, src_hbm, dst_hbm, out_hbm,
              acc, src_v, dst_v, *ring):
    rows = ring[:NBUF]
    gsem = ring[NBUF:2 * NBUF]
    ssem = ring[2 * NBUF:3 * NBUF]
    cid = lax.axis_index("c")
    sid = lax.axis_index("s")
    wid = sid * NC + cid

    r0 = sid * RPT
    for k, ch in enumerate(IOCH):
        pltpu.sync_copy(z_hbm.at[pl.ds(r0, ch)], rows[k].at[pl.ds(0, ch)])
        pltpu.sync_copy(rows[k].at[pl.ds(0, ch)], acc.at[pl.ds(r0, ch)])
        r0 = r0 + ch
    plsc.subcore_barrier()

    pltpu.sync_copy(src_hbm.at[wid], src_v)
    pltpu.sync_copy(dst_hbm.at[wid], dst_v)

    for b in range(H):
        pltpu.async_copy(h_hbm.at[src_v.at[b]], rows[b], gsem[b])

    def group(g, c):
        for b in range(NBUF):
            j = g * NBUF + b
            pltpu.make_async_copy(
                h_hbm.at[src_v.at[j]], rows[b], gsem[b]).wait()
            pltpu.async_copy(rows[b], acc.at[dst_v.at[j]], ssem[b], add=True)
            h = (b + H) % NBUF
            jg = j + H

            @pl.when(jg < KN)
            def _prefetch(h=h, jg=jg):
                @pl.when(jg >= NBUF)
                def _wait_scatter():
                    pltpu.make_async_copy(
                        rows[h], acc.at[dst_v.at[jg - NBUF]], ssem[h]).wait()
                pltpu.async_copy(h_hbm.at[src_v.at[jg]], rows[h], gsem[h])
        return c
    lax.fori_loop(0, NG, group, 0)
    for b in range(NBUF):
        pltpu.make_async_copy(
            rows[b], acc.at[dst_v.at[KN - NBUF + b]], ssem[b]).wait()
    plsc.subcore_barrier()

    r1 = sid * RPT
    for k, ch in enumerate(IOCH):
        pltpu.sync_copy(acc.at[pl.ds(r1, ch)], rows[k].at[pl.ds(0, ch)])
        pltpu.sync_copy(rows[k].at[pl.ds(0, ch)], out_hbm.at[cid, pl.ds(r1, ch)])
        r1 = r1 + ch


def _make_agg(d):
    return functools.partial(
        pl.kernel,
        out_type=jax.ShapeDtypeStruct((NC, N, d), jnp.float32),
        mesh=_mesh(),
        compiler_params=_SC_PARAMS,
        scratch_types=(
            [pltpu.VMEM_SHARED((N + NDUMP, d), jnp.float32),
             pltpu.VMEM((KN, KB), jnp.int32),
             pltpu.VMEM((KN, KB), jnp.int32)]
            + [pltpu.VMEM((KB, d), jnp.float32) for _ in range(NBUF)]
            + [pltpu.SemaphoreType.DMA for _ in range(2 * NBUF)]
        ))(_agg_body)


_agg64 = _make_agg(D_HID)
_agg16 = _make_agg(D_OUT)


BR = 2000


def _tc1_body(x_ref, w_ref, deg_ref, h_ref, dinv_ref):
    dinv = lax.rsqrt(deg_ref[...] + 1.0)
    h = jnp.dot(x_ref[...], w_ref[...], preferred_element_type=jnp.float32)
    h_ref[...] = h * dinv
    dinv_ref[...] = dinv


_tc1 = pl.pallas_call(
    _tc1_body,
    grid=(N // BR,),
    in_specs=[pl.BlockSpec((BR, D_IN), lambda i: (i, 0)),
              pl.BlockSpec((D_IN, D_HID), lambda i: (0, 0)),
              pl.BlockSpec((BR, 1), lambda i: (i, 0))],
    out_specs=[pl.BlockSpec((BR, D_HID), lambda i: (i, 0)),
               pl.BlockSpec((BR, 1), lambda i: (i, 0))],
    out_shape=[jax.ShapeDtypeStruct((N, D_HID), jnp.float32),
               jax.ShapeDtypeStruct((N, 1), jnp.float32)])


def _tc_mid_body(p_ref, hp_ref, dinv_ref, b_ref, w_ref, out_ref):
    t = p_ref[0] + p_ref[1] + hp_ref[...]
    t = t * dinv_ref[...] + b_ref[...]
    t = jnp.maximum(t, 0.0)
    out_ref[...] = jnp.dot(
        t, w_ref[...], preferred_element_type=jnp.float32) * dinv_ref[...]


_tc_mid = pl.pallas_call(
    _tc_mid_body,
    grid=(N // BR,),
    in_specs=[pl.BlockSpec((NC, BR, D_HID), lambda i: (0, i, 0)),
              pl.BlockSpec((BR, D_HID), lambda i: (i, 0)),
              pl.BlockSpec((BR, 1), lambda i: (i, 0)),
              pl.BlockSpec((1, D_HID), lambda i: (0, 0)),
              pl.BlockSpec((D_HID, D_OUT), lambda i: (0, 0))],
    out_specs=pl.BlockSpec((BR, D_OUT), lambda i: (i, 0)),
    out_shape=jax.ShapeDtypeStruct((N, D_OUT), jnp.float32))


def _tc_out_body(p_ref, hp_ref, dinv_ref, b_ref, out_ref):
    t = (p_ref[0] + p_ref[1] + hp_ref[...]) * dinv_ref[...] + b_ref[...]
    m = jnp.max(t, axis=1, keepdims=True)
    e = jnp.exp(t - m)
    s = jnp.sum(e, axis=1, keepdims=True)
    out_ref[...] = (t - m) - jnp.log(s)


_tc_out = pl.pallas_call(
    _tc_out_body,
    grid=(N // BR,),
    in_specs=[pl.BlockSpec((NC, BR, D_OUT), lambda i: (0, i, 0)),
              pl.BlockSpec((BR, D_OUT), lambda i: (i, 0)),
              pl.BlockSpec((BR, 1), lambda i: (i, 0)),
              pl.BlockSpec((1, D_OUT), lambda i: (0, 0))],
    out_specs=pl.BlockSpec((BR, D_OUT), lambda i: (i, 0)),
    out_shape=jax.ShapeDtypeStruct((N, D_OUT), jnp.float32))


def kernel(x, edge_index, W1, b1, W2, b2):
    ei = edge_index.astype(jnp.int32)
    pad = jnp.arange(EPAD, dtype=jnp.int32)
    src3 = jnp.concatenate([ei[0], (pad * 8) % N]).reshape(NW, KN, KB)
    dst3 = jnp.concatenate([ei[1], N + (pad % NDUMP)]).reshape(NW, KN, KB)
    z1 = jnp.zeros((N,), jnp.float32)
    z64 = jnp.zeros((N, D_HID), jnp.float32)
    z16 = jnp.zeros((N, D_OUT), jnp.float32)

    degp = _deg(dst3, z1).reshape(NC, N)
    deg = (degp[0] + degp[1]).reshape(N, 1)
    h1p, dinv = _tc1(x, W1, deg)
    p1 = _agg64(h1p, z64, src3, dst3)
    h2p = _tc_mid(p1, h1p, dinv, b1.reshape(1, D_HID), W2)
    p2 = _agg16(h2p, z16, src3, dst3)
    return _tc_out(p2, h2p, dinv, b2.reshape(1, D_OUT))


# speedup vs baseline: 1.2379x; 1.0198x over previous
import functools

import jax
import jax.numpy as jnp
from jax import lax
from jax.experimental import pallas as pl
from jax.experimental.pallas import tpu as pltpu
from jax.experimental.pallas import tpu_sc as plsc

N = 10000
E = 320000
D_IN, D_HID, D_OUT = 128, 64, 16

NC, NS = 2, 16
NW = NC * NS
KB = 128
EPAD = 7680
EP = E + EPAD
KN = EP // (NW * KB)
NBUF = 8
H = 7
NG = KN // NBUF
DFL = 20
NDUMP = 8
RPT = N // NS
IOCH = (KB, KB, KB, KB, RPT - 4 * KB)


def _mesh():
    return plsc.VectorSubcoreMesh(
        core_axis_name="c", subcore_axis_name="s",
        num_cores=NC, num_subcores=NS)


_SC_PARAMS = pltpu.CompilerParams(use_tc_tiling_on_sc=False)


def _deg_body(dst_hbm, z1_hbm, out_hbm, acc, dst_v, ones_v, obuf, dsem):
    cid = lax.axis_index("c")
    sid = lax.axis_index("s")
    wid = sid * NC + cid

    @pl.when(sid < 10)
    def _init():
        pltpu.sync_copy(z1_hbm.at[pl.ds(sid * 1000, 1000)], obuf)
        pltpu.sync_copy(obuf, acc.at[pl.ds(sid * 1000, 1000)])

    for i in range(KB // 16):
        ones_v[pl.ds(i * 16, 16)] = jnp.ones((16,), jnp.float32)
    plsc.subcore_barrier()

    pltpu.sync_copy(dst_hbm.at[wid], dst_v)
    ones = ones_v

    def group(g, c):
        for i in range(DFL):
            pltpu.async_copy(ones, acc.at[dst_v.at[g * DFL + i]],
                             dsem, add=True)
        for i in range(DFL):
            pltpu.make_async_copy(
                ones, acc.at[dst_v.at[g * DFL + i]], dsem).wait()
        return c
    lax.fori_loop(0, KN // DFL, group, 0)
    plsc.subcore_barrier()

    @pl.when(sid < 10)
    def _out():
        pltpu.sync_copy(acc.at[pl.ds(sid * 1000, 1000)], obuf)
        pltpu.sync_copy(obuf, out_hbm.at[pl.ds(cid * N + sid * 1000, 1000)])


_deg = functools.partial(
    pl.kernel,
    out_type=jax.ShapeDtypeStruct((NC * N,), jnp.float32),
    mesh=_mesh(),
    compiler_params=_SC_PARAMS,
    scratch_types=[
        pltpu.VMEM_SHARED((N + NDUMP,), jnp.float32),
        pltpu.VMEM((KN, KB), jnp.int32),
        pltpu.VMEM((KB,), jnp.float32),
        pltpu.VMEM((1000,), jnp.float32),
        pltpu.SemaphoreType.DMA,
    ])(_deg_body)


def _agg_body(h_hbm, z_hbm, src_hbm, dst_hbm, out_hbm,
              acc, src_v, dst_v, *ring):
    rows = ring[:NBUF]
    gsem = ring[NBUF:2 * NBUF]
    ssem = ring[2 * NBUF:3 * NBUF]
    cid = lax.axis_index("c")
    sid = lax.axis_index("s")
    wid = sid * NC + cid

    r0 = sid * RPT
    for k, ch in enumerate(IOCH):
        pltpu.sync_copy(z_hbm.at[pl.ds(r0, ch)], rows[k].at[pl.ds(0, ch)])
        pltpu.sync_copy(rows[k].at[pl.ds(0, ch)], acc.at[pl.ds(r0, ch)])
        r0 = r0 + ch
    plsc.subcore_barrier()

    pltpu.sync_copy(src_hbm.at[wid], src_v)
    pltpu.sync_copy(dst_hbm.at[wid], dst_v)

    for b in range(H):
        pltpu.async_copy(h_hbm.at[src_v.at[b]], rows[b], gsem[b])

    def group(g, c):
        for b in range(NBUF):
            j = g * NBUF + b
            pltpu.make_async_copy(
                h_hbm.at[src_v.at[j]], rows[b], gsem[b]).wait()
            pltpu.async_copy(rows[b], acc.at[dst_v.at[j]], ssem[b], add=True)
            h = (b + H) % NBUF
            jg = j + H

            @pl.when(jg < KN)
            def _prefetch(h=h, jg=jg):
                @pl.when(jg >= NBUF)
                def _wait_scatter():
                    pltpu.make_async_copy(
                        rows[h], acc.at[dst_v.at[jg - NBUF]], ssem[h]).wait()
                pltpu.async_copy(h_hbm.at[src_v.at[jg]], rows[h], gsem[h])
        return c
    lax.fori_loop(0, NG, group, 0)
    for b in range(NBUF):
        pltpu.make_async_copy(
            rows[b], acc.at[dst_v.at[KN - NBUF + b]], ssem[b]).wait()
    plsc.subcore_barrier()

    r1 = sid * RPT
    for k, ch in enumerate(IOCH):
        pltpu.sync_copy(acc.at[pl.ds(r1, ch)], rows[k].at[pl.ds(0, ch)])
        pltpu.sync_copy(rows[k].at[pl.ds(0, ch)], out_hbm.at[cid, pl.ds(r1, ch)])
        r1 = r1 + ch


def _make_agg(d):
    return functools.partial(
        pl.kernel,
        out_type=jax.ShapeDtypeStruct((NC, N, d), jnp.float32),
        mesh=_mesh(),
        compiler_params=_SC_PARAMS,
        scratch_types=(
            [pltpu.VMEM_SHARED((N + NDUMP, d), jnp.float32),
             pltpu.VMEM((KN, KB), jnp.int32),
             pltpu.VMEM((KN, KB), jnp.int32)]
            + [pltpu.VMEM((KB, d), jnp.float32) for _ in range(NBUF)]
            + [pltpu.SemaphoreType.DMA for _ in range(2 * NBUF)]
        ))(_agg_body)


_agg64 = _make_agg(D_HID)
_agg16 = _make_agg(D_OUT)


BR = 5000


def _tc1_body(x_ref, w_ref, deg_ref, h_ref, dinv_ref):
    dinv = lax.rsqrt(deg_ref[...] + 1.0)
    h = jnp.dot(x_ref[...], w_ref[...], preferred_element_type=jnp.float32)
    h_ref[...] = h * dinv
    dinv_ref[...] = dinv


_tc1 = pl.pallas_call(
    _tc1_body,
    grid=(N // BR,),
    in_specs=[pl.BlockSpec((BR, D_IN), lambda i: (i, 0)),
              pl.BlockSpec((D_IN, D_HID), lambda i: (0, 0)),
              pl.BlockSpec((BR, 1), lambda i: (i, 0))],
    out_specs=[pl.BlockSpec((BR, D_HID), lambda i: (i, 0)),
               pl.BlockSpec((BR, 1), lambda i: (i, 0))],
    out_shape=[jax.ShapeDtypeStruct((N, D_HID), jnp.float32),
               jax.ShapeDtypeStruct((N, 1), jnp.float32)])


def _tc_mid_body(p_ref, hp_ref, dinv_ref, b_ref, w_ref, out_ref):
    t = p_ref[0] + p_ref[1] + hp_ref[...]
    t = t * dinv_ref[...] + b_ref[...]
    t = jnp.maximum(t, 0.0)
    out_ref[...] = jnp.dot(
        t, w_ref[...], preferred_element_type=jnp.float32) * dinv_ref[...]


_tc_mid = pl.pallas_call(
    _tc_mid_body,
    grid=(N // BR,),
    in_specs=[pl.BlockSpec((NC, BR, D_HID), lambda i: (0, i, 0)),
              pl.BlockSpec((BR, D_HID), lambda i: (i, 0)),
              pl.BlockSpec((BR, 1), lambda i: (i, 0)),
              pl.BlockSpec((1, D_HID), lambda i: (0, 0)),
              pl.BlockSpec((D_HID, D_OUT), lambda i: (0, 0))],
    out_specs=pl.BlockSpec((BR, D_OUT), lambda i: (i, 0)),
    out_shape=jax.ShapeDtypeStruct((N, D_OUT), jnp.float32))


def _tc_out_body(p_ref, hp_ref, dinv_ref, b_ref, out_ref):
    t = (p_ref[0] + p_ref[1] + hp_ref[...]) * dinv_ref[...] + b_ref[...]
    m = jnp.max(t, axis=1, keepdims=True)
    e = jnp.exp(t - m)
    s = jnp.sum(e, axis=1, keepdims=True)
    out_ref[...] = (t - m) - jnp.log(s)


_tc_out = pl.pallas_call(
    _tc_out_body,
    grid=(N // BR,),
    in_specs=[pl.BlockSpec((NC, BR, D_OUT), lambda i: (0, i, 0)),
              pl.BlockSpec((BR, D_OUT), lambda i: (i, 0)),
              pl.BlockSpec((BR, 1), lambda i: (i, 0)),
              pl.BlockSpec((1, D_OUT), lambda i: (0, 0))],
    out_specs=pl.BlockSpec((BR, D_OUT), lambda i: (i, 0)),
    out_shape=jax.ShapeDtypeStruct((N, D_OUT), jnp.float32))


def kernel(x, edge_index, W1, b1, W2, b2):
    ei = edge_index.astype(jnp.int32)
    pad = jnp.arange(EPAD, dtype=jnp.int32)
    src3 = jnp.concatenate([ei[0], (pad * 8) % N]).reshape(NW, KN, KB)
    dst3 = jnp.concatenate([ei[1], N + (pad % NDUMP)]).reshape(NW, KN, KB)
    z1 = jnp.zeros((N,), jnp.float32)
    z64 = jnp.zeros((N, D_HID), jnp.float32)
    z16 = jnp.zeros((N, D_OUT), jnp.float32)

    degp = _deg(dst3, z1).reshape(NC, N)
    deg = (degp[0] + degp[1]).reshape(N, 1)
    h1p, dinv = _tc1(x, W1, deg)
    p1 = _agg64(h1p, z64, src3, dst3)
    h2p = _tc_mid(p1, h1p, dinv, b1.reshape(1, D_HID), W2)
    p2 = _agg16(h2p, z16, src3, dst3)
    return _tc_out(p2, h2p, dinv, b2.reshape(1, D_OUT))


# speedup vs baseline: 1.2909x; 1.0428x over previous
import functools

import jax
import jax.numpy as jnp
from jax import lax
from jax.experimental import pallas as pl
from jax.experimental.pallas import tpu as pltpu
from jax.experimental.pallas import tpu_sc as plsc

N = 10000
E = 320000
D_IN, D_HID, D_OUT = 128, 64, 16

NC, NS = 2, 16
NW = NC * NS
KB = 128
EPAD = 7680
EP = E + EPAD
KN = EP // (NW * KB)
NBUF = 8
H = 7
NG = KN // NBUF
DFL = 20
NDUMP = 8
RPT = N // NS
IOCH = (KB, KB, KB, KB, RPT - 4 * KB)


def _mesh():
    return plsc.VectorSubcoreMesh(
        core_axis_name="c", subcore_axis_name="s",
        num_cores=NC, num_subcores=NS)


_SC_PARAMS = pltpu.CompilerParams(use_tc_tiling_on_sc=False)


def _deg_body(dst_hbm, z1_hbm, out_hbm, acc, dst_v, ones_v, obuf, dsem):
    cid = lax.axis_index("c")
    sid = lax.axis_index("s")
    wid = sid * NC + cid

    @pl.when(sid < 10)
    def _init():
        pltpu.sync_copy(z1_hbm.at[pl.ds(sid * 1000, 1000)], obuf)
        pltpu.sync_copy(obuf, acc.at[pl.ds(sid * 1000, 1000)])

    for i in range(KB // 16):
        ones_v[pl.ds(i * 16, 16)] = jnp.ones((16,), jnp.float32)
    plsc.subcore_barrier()

    pltpu.sync_copy(dst_hbm.at[1, wid], dst_v)
    ones = ones_v

    def group(g, c):
        for i in range(DFL):
            pltpu.async_copy(ones, acc.at[dst_v.at[g * DFL + i]],
                             dsem, add=True)
        for i in range(DFL):
            pltpu.make_async_copy(
                ones, acc.at[dst_v.at[g * DFL + i]], dsem).wait()
        return c
    lax.fori_loop(0, KN // DFL, group, 0)
    plsc.subcore_barrier()

    @pl.when(sid < 10)
    def _out():
        pltpu.sync_copy(acc.at[pl.ds(sid * 1000, 1000)], obuf)
        pltpu.sync_copy(obuf, out_hbm.at[pl.ds(cid * N + sid * 1000, 1000)])


_deg = functools.partial(
    pl.kernel,
    out_type=jax.ShapeDtypeStruct((NC * N,), jnp.float32),
    mesh=_mesh(),
    compiler_params=_SC_PARAMS,
    scratch_types=[
        pltpu.VMEM_SHARED((N + NDUMP,), jnp.float32),
        pltpu.VMEM((KN, KB), jnp.int32),
        pltpu.VMEM((KB,), jnp.float32),
        pltpu.VMEM((1000,), jnp.float32),
        pltpu.SemaphoreType.DMA,
    ])(_deg_body)


def _agg_body(h_hbm, z_hbm, e_hbm, out_hbm,
              acc, src_v, dst_v, *ring):
    rows = ring[:NBUF]
    gsem = ring[NBUF:2 * NBUF]
    ssem = ring[2 * NBUF:3 * NBUF]
    cid = lax.axis_index("c")
    sid = lax.axis_index("s")
    wid = sid * NC + cid

    r0 = sid * RPT
    for k, ch in enumerate(IOCH):
        pltpu.sync_copy(z_hbm.at[pl.ds(r0, ch)], rows[k].at[pl.ds(0, ch)])
        pltpu.sync_copy(rows[k].at[pl.ds(0, ch)], acc.at[pl.ds(r0, ch)])
        r0 = r0 + ch
    plsc.subcore_barrier()

    pltpu.sync_copy(e_hbm.at[0, wid], src_v)
    pltpu.sync_copy(e_hbm.at[1, wid], dst_v)

    for b in range(H):
        pltpu.async_copy(h_hbm.at[src_v.at[b]], rows[b], gsem[b])

    def group(g, c):
        for b in range(NBUF):
            j = g * NBUF + b
            pltpu.make_async_copy(
                h_hbm.at[src_v.at[j]], rows[b], gsem[b]).wait()
            pltpu.async_copy(rows[b], acc.at[dst_v.at[j]], ssem[b], add=True)
            h = (b + H) % NBUF
            jg = j + H

            @pl.when(jg < KN)
            def _prefetch(h=h, jg=jg):
                @pl.when(jg >= NBUF)
                def _wait_scatter():
                    pltpu.make_async_copy(
                        rows[h], acc.at[dst_v.at[jg - NBUF]], ssem[h]).wait()
                pltpu.async_copy(h_hbm.at[src_v.at[jg]], rows[h], gsem[h])
        return c
    lax.fori_loop(0, NG, group, 0)
    for b in range(NBUF):
        pltpu.make_async_copy(
            rows[b], acc.at[dst_v.at[KN - NBUF + b]], ssem[b]).wait()
    plsc.subcore_barrier()

    r1 = sid * RPT
    for k, ch in enumerate(IOCH):
        pltpu.sync_copy(acc.at[pl.ds(r1, ch)], rows[k].at[pl.ds(0, ch)])
        pltpu.sync_copy(rows[k].at[pl.ds(0, ch)], out_hbm.at[cid, pl.ds(r1, ch)])
        r1 = r1 + ch


def _make_agg(d):
    return functools.partial(
        pl.kernel,
        out_type=jax.ShapeDtypeStruct((NC, N, d), jnp.float32),
        mesh=_mesh(),
        compiler_params=_SC_PARAMS,
        scratch_types=(
            [pltpu.VMEM_SHARED((N + NDUMP, d), jnp.float32),
             pltpu.VMEM((KN, KB), jnp.int32),
             pltpu.VMEM((KN, KB), jnp.int32)]
            + [pltpu.VMEM((KB, d), jnp.float32) for _ in range(NBUF)]
            + [pltpu.SemaphoreType.DMA for _ in range(2 * NBUF)]
        ))(_agg_body)


_agg64 = _make_agg(D_HID)
_agg16 = _make_agg(D_OUT)


BR = 5000


def _tc1_body(x_ref, w_ref, deg_ref, h_ref, dinv_ref):
    dinv = lax.rsqrt(deg_ref[...] + 1.0)
    h = jnp.dot(x_ref[...], w_ref[...], preferred_element_type=jnp.float32)
    h_ref[...] = h * dinv
    dinv_ref[...] = dinv


_tc1 = pl.pallas_call(
    _tc1_body,
    grid=(N // BR,),
    in_specs=[pl.BlockSpec((BR, D_IN), lambda i: (i, 0)),
              pl.BlockSpec((D_IN, D_HID), lambda i: (0, 0)),
              pl.BlockSpec((BR, 1), lambda i: (i, 0))],
    out_specs=[pl.BlockSpec((BR, D_HID), lambda i: (i, 0)),
               pl.BlockSpec((BR, 1), lambda i: (i, 0))],
    out_shape=[jax.ShapeDtypeStruct((N, D_HID), jnp.float32),
               jax.ShapeDtypeStruct((N, 1), jnp.float32)])


def _tc_mid_body(p_ref, hp_ref, dinv_ref, b_ref, w_ref, out_ref):
    t = p_ref[0] + p_ref[1] + hp_ref[...]
    t = t * dinv_ref[...] + b_ref[...]
    t = jnp.maximum(t, 0.0)
    out_ref[...] = jnp.dot(
        t, w_ref[...], preferred_element_type=jnp.float32) * dinv_ref[...]


_tc_mid = pl.pallas_call(
    _tc_mid_body,
    grid=(N // BR,),
    in_specs=[pl.BlockSpec((NC, BR, D_HID), lambda i: (0, i, 0)),
              pl.BlockSpec((BR, D_HID), lambda i: (i, 0)),
              pl.BlockSpec((BR, 1), lambda i: (i, 0)),
              pl.BlockSpec((1, D_HID), lambda i: (0, 0)),
              pl.BlockSpec((D_HID, D_OUT), lambda i: (0, 0))],
    out_specs=pl.BlockSpec((BR, D_OUT), lambda i: (i, 0)),
    out_shape=jax.ShapeDtypeStruct((N, D_OUT), jnp.float32))


def _tc_out_body(p_ref, hp_ref, dinv_ref, b_ref, out_ref):
    t = (p_ref[0] + p_ref[1] + hp_ref[...]) * dinv_ref[...] + b_ref[...]
    m = jnp.max(t, axis=1, keepdims=True)
    e = jnp.exp(t - m)
    s = jnp.sum(e, axis=1, keepdims=True)
    out_ref[...] = (t - m) - jnp.log(s)


_tc_out = pl.pallas_call(
    _tc_out_body,
    grid=(N // BR,),
    in_specs=[pl.BlockSpec((NC, BR, D_OUT), lambda i: (0, i, 0)),
              pl.BlockSpec((BR, D_OUT), lambda i: (i, 0)),
              pl.BlockSpec((BR, 1), lambda i: (i, 0)),
              pl.BlockSpec((1, D_OUT), lambda i: (0, 0))],
    out_specs=pl.BlockSpec((BR, D_OUT), lambda i: (i, 0)),
    out_shape=jax.ShapeDtypeStruct((N, D_OUT), jnp.float32))


def kernel(x, edge_index, W1, b1, W2, b2):
    ei = edge_index.astype(jnp.int32)
    pad = jnp.arange(EPAD, dtype=jnp.int32)
    epad = jnp.stack([(pad * 8) % N, N + (pad % NDUMP)])
    eslab = jnp.concatenate([ei, epad], axis=1).reshape(2, NW, KN, KB)
    z1 = jnp.zeros((N,), jnp.float32)
    z64 = jnp.zeros((N, D_HID), jnp.float32)
    z16 = jnp.zeros((N, D_OUT), jnp.float32)

    degp = _deg(eslab, z1).reshape(NC, N)
    deg = (degp[0] + degp[1]).reshape(N, 1)
    h1p, dinv = _tc1(x, W1, deg)
    p1 = _agg64(h1p, z64, eslab)
    h2p = _tc_mid(p1, h1p, dinv, b1.reshape(1, D_HID), W2)
    p2 = _agg16(h2p, z16, eslab)
    return _tc_out(p2, h2p, dinv, b2.reshape(1, D_OUT))


# speedup vs baseline: 1.3774x; 1.0670x over previous
import functools

import jax
import jax.numpy as jnp
from jax import lax
from jax.experimental import pallas as pl
from jax.experimental.pallas import tpu as pltpu
from jax.experimental.pallas import tpu_sc as plsc

N = 10000
E = 320000
D_IN, D_HID, D_OUT = 128, 64, 16

NC, NS = 2, 16
NW = NC * NS
KB = 128
EPAD = 7680
EP = E + EPAD
KN = EP // (NW * KB)
NBUF = 8
H = 7
NG = KN // NBUF
DFL = 20
NDUMP = 8
RPT = N // NS
IOCH = (KB, KB, KB, KB, RPT - 4 * KB)


def _mesh():
    return plsc.VectorSubcoreMesh(
        core_axis_name="c", subcore_axis_name="s",
        num_cores=NC, num_subcores=NS)


_SC_PARAMS = pltpu.CompilerParams(use_tc_tiling_on_sc=False)


def _deg_body(dst_hbm, out_hbm, acc, dst_v, ones_v, obuf, dsem):
    cid = lax.axis_index("c")
    sid = lax.axis_index("s")
    wid = sid * NC + cid

    def zfill(i, c):
        obuf[pl.ds(i * 16, 16)] = jnp.zeros((16,), jnp.float32)
        return c
    lax.fori_loop(0, 1000 // 16 + 1, zfill, 0)

    @pl.when(sid < 10)
    def _init():
        pltpu.sync_copy(obuf.at[pl.ds(0, 1000)],
                        acc.at[pl.ds(sid * 1000, 1000)])

    for i in range(KB // 16):
        ones_v[pl.ds(i * 16, 16)] = jnp.ones((16,), jnp.float32)
    plsc.subcore_barrier()

    pltpu.sync_copy(dst_hbm.at[1, wid], dst_v)
    ones = ones_v

    def group(g, c):
        for i in range(DFL):
            pltpu.async_copy(ones, acc.at[dst_v.at[g * DFL + i]],
                             dsem, add=True)
        for i in range(DFL):
            pltpu.make_async_copy(
                ones, acc.at[dst_v.at[g * DFL + i]], dsem).wait()
        return c
    lax.fori_loop(0, KN // DFL, group, 0)
    plsc.subcore_barrier()

    @pl.when(sid < 10)
    def _out():
        pltpu.sync_copy(acc.at[pl.ds(sid * 1000, 1000)],
                        obuf.at[pl.ds(0, 1000)])
        pltpu.sync_copy(obuf.at[pl.ds(0, 1000)],
                        out_hbm.at[pl.ds(cid * N + sid * 1000, 1000)])


_deg = functools.partial(
    pl.kernel,
    out_type=jax.ShapeDtypeStruct((NC * N,), jnp.float32),
    mesh=_mesh(),
    compiler_params=_SC_PARAMS,
    scratch_types=[
        pltpu.VMEM_SHARED((N + NDUMP,), jnp.float32),
        pltpu.VMEM((KN, KB), jnp.int32),
        pltpu.VMEM((KB,), jnp.float32),
        pltpu.VMEM((1008,), jnp.float32),
        pltpu.SemaphoreType.DMA,
    ])(_deg_body)


def _agg_body(h_hbm, e_hbm, out_hbm, acc, src_v, dst_v, *ring):
    rows = ring[:NBUF]
    gsem = ring[NBUF:2 * NBUF]
    ssem = ring[2 * NBUF:3 * NBUF]
    cid = lax.axis_index("c")
    sid = lax.axis_index("s")
    wid = sid * NC + cid
    d = rows[0].shape[1]

    def zfill(i, c):
        for j in range(d // 16):
            rows[0][i, pl.ds(j * 16, 16)] = jnp.zeros((16,), jnp.float32)
        return c
    lax.fori_loop(0, KB, zfill, 0)
    r0 = sid * RPT
    for ch in IOCH:
        pltpu.sync_copy(rows[0].at[pl.ds(0, ch)], acc.at[pl.ds(r0, ch)])
        r0 = r0 + ch
    plsc.subcore_barrier()

    pltpu.sync_copy(e_hbm.at[0, wid], src_v)
    pltpu.sync_copy(e_hbm.at[1, wid], dst_v)

    for b in range(H):
        pltpu.async_copy(h_hbm.at[src_v.at[b]], rows[b], gsem[b])

    def group(g, c):
        for b in range(NBUF):
            j = g * NBUF + b
            pltpu.make_async_copy(
                h_hbm.at[src_v.at[j]], rows[b], gsem[b]).wait()
            pltpu.async_copy(rows[b], acc.at[dst_v.at[j]], ssem[b], add=True)
            h = (b + H) % NBUF
            jg = j + H

            @pl.when(jg < KN)
            def _prefetch(h=h, jg=jg):
                @pl.when(jg >= NBUF)
                def _wait_scatter():
                    pltpu.make_async_copy(
                        rows[h], acc.at[dst_v.at[jg - NBUF]], ssem[h]).wait()
                pltpu.async_copy(h_hbm.at[src_v.at[jg]], rows[h], gsem[h])
        return c
    lax.fori_loop(0, NG, group, 0)
    for b in range(NBUF):
        pltpu.make_async_copy(
            rows[b], acc.at[dst_v.at[KN - NBUF + b]], ssem[b]).wait()
    plsc.subcore_barrier()

    r1 = sid * RPT
    for k, ch in enumerate(IOCH):
        pltpu.sync_copy(acc.at[pl.ds(r1, ch)], rows[k].at[pl.ds(0, ch)])
        pltpu.sync_copy(rows[k].at[pl.ds(0, ch)], out_hbm.at[cid, pl.ds(r1, ch)])
        r1 = r1 + ch


def _make_agg(d):
    return functools.partial(
        pl.kernel,
        out_type=jax.ShapeDtypeStruct((NC, N, d), jnp.float32),
        mesh=_mesh(),
        compiler_params=_SC_PARAMS,
        scratch_types=(
            [pltpu.VMEM_SHARED((N + NDUMP, d), jnp.float32),
             pltpu.VMEM((KN, KB), jnp.int32),
             pltpu.VMEM((KN, KB), jnp.int32)]
            + [pltpu.VMEM((KB, d), jnp.float32) for _ in range(NBUF)]
            + [pltpu.SemaphoreType.DMA for _ in range(2 * NBUF)]
        ))(_agg_body)


_agg64 = _make_agg(D_HID)
_agg16 = _make_agg(D_OUT)


BR = 5000


def _tc1_body(x_ref, w_ref, deg_ref, h_ref, dinv_ref):
    dinv = lax.rsqrt(deg_ref[...] + 1.0)
    h = jnp.dot(x_ref[...], w_ref[...], preferred_element_type=jnp.float32)
    h_ref[...] = h * dinv
    dinv_ref[...] = dinv


_tc1 = pl.pallas_call(
    _tc1_body,
    grid=(N // BR,),
    in_specs=[pl.BlockSpec((BR, D_IN), lambda i: (i, 0)),
              pl.BlockSpec((D_IN, D_HID), lambda i: (0, 0)),
              pl.BlockSpec((BR, 1), lambda i: (i, 0))],
    out_specs=[pl.BlockSpec((BR, D_HID), lambda i: (i, 0)),
               pl.BlockSpec((BR, 1), lambda i: (i, 0))],
    out_shape=[jax.ShapeDtypeStruct((N, D_HID), jnp.float32),
               jax.ShapeDtypeStruct((N, 1), jnp.float32)])


def _tc_mid_body(p_ref, hp_ref, dinv_ref, b_ref, w_ref, out_ref):
    t = p_ref[0] + p_ref[1] + hp_ref[...]
    t = t * dinv_ref[...] + b_ref[...]
    t = jnp.maximum(t, 0.0)
    out_ref[...] = jnp.dot(
        t, w_ref[...], preferred_element_type=jnp.float32) * dinv_ref[...]


_tc_mid = pl.pallas_call(
    _tc_mid_body,
    grid=(N // BR,),
    in_specs=[pl.BlockSpec((NC, BR, D_HID), lambda i: (0, i, 0)),
              pl.BlockSpec((BR, D_HID), lambda i: (i, 0)),
              pl.BlockSpec((BR, 1), lambda i: (i, 0)),
              pl.BlockSpec((1, D_HID), lambda i: (0, 0)),
              pl.BlockSpec((D_HID, D_OUT), lambda i: (0, 0))],
    out_specs=pl.BlockSpec((BR, D_OUT), lambda i: (i, 0)),
    out_shape=jax.ShapeDtypeStruct((N, D_OUT), jnp.float32))


def _tc_out_body(p_ref, hp_ref, dinv_ref, b_ref, out_ref):
    t = (p_ref[0] + p_ref[1] + hp_ref[...]) * dinv_ref[...] + b_ref[...]
    m = jnp.max(t, axis=1, keepdims=True)
    e = jnp.exp(t - m)
    s = jnp.sum(e, axis=1, keepdims=True)
    out_ref[...] = (t - m) - jnp.log(s)


_tc_out = pl.pallas_call(
    _tc_out_body,
    grid=(N // BR,),
    in_specs=[pl.BlockSpec((NC, BR, D_OUT), lambda i: (0, i, 0)),
              pl.BlockSpec((BR, D_OUT), lambda i: (i, 0)),
              pl.BlockSpec((BR, 1), lambda i: (i, 0)),
              pl.BlockSpec((1, D_OUT), lambda i: (0, 0))],
    out_specs=pl.BlockSpec((BR, D_OUT), lambda i: (i, 0)),
    out_shape=jax.ShapeDtypeStruct((N, D_OUT), jnp.float32))


def kernel(x, edge_index, W1, b1, W2, b2):
    ei = edge_index.astype(jnp.int32)
    pad = jnp.arange(EPAD, dtype=jnp.int32)
    epad = jnp.stack([(pad * 8) % N, N + (pad % NDUMP)])
    eslab = jnp.concatenate([ei, epad], axis=1).reshape(2, NW, KN, KB)
    degp = _deg(eslab).reshape(NC, N)
    deg = (degp[0] + degp[1]).reshape(N, 1)
    h1p, dinv = _tc1(x, W1, deg)
    p1 = _agg64(h1p, eslab)
    h2p = _tc_mid(p1, h1p, dinv, b1.reshape(1, D_HID), W2)
    p2 = _agg16(h2p, eslab)
    return _tc_out(p2, h2p, dinv, b2.reshape(1, D_OUT))


# speedup vs baseline: 1.3778x; 1.0003x over previous
import functools

import jax
import jax.numpy as jnp
from jax import lax
from jax.experimental import pallas as pl
from jax.experimental.pallas import tpu as pltpu
from jax.experimental.pallas import tpu_sc as plsc

N = 10000
E = 320000
D_IN, D_HID, D_OUT = 128, 64, 16

NC, NS = 2, 16
NW = NC * NS
KB = 128
EPAD = 7680
EP = E + EPAD
KN = EP // (NW * KB)
DFL = 40
NDUMP = 8
RPT = N // NS
IOCH = (KB, KB, KB, KB, RPT - 4 * KB)


def _mesh():
    return plsc.VectorSubcoreMesh(
        core_axis_name="c", subcore_axis_name="s",
        num_cores=NC, num_subcores=NS)


_SC_PARAMS = pltpu.CompilerParams(use_tc_tiling_on_sc=False)


def _deg_body(dst_hbm, out_hbm, acc, dst_v, ones_v, obuf, dsem):
    cid = lax.axis_index("c")
    sid = lax.axis_index("s")
    wid = sid * NC + cid

    def zfill(i, c):
        obuf[pl.ds(i * 16, 16)] = jnp.zeros((16,), jnp.float32)
        return c
    lax.fori_loop(0, 1000 // 16 + 1, zfill, 0)

    @pl.when(sid < 10)
    def _init():
        pltpu.sync_copy(obuf.at[pl.ds(0, 1000)],
                        acc.at[pl.ds(sid * 1000, 1000)])

    for i in range(KB // 16):
        ones_v[pl.ds(i * 16, 16)] = jnp.ones((16,), jnp.float32)
    plsc.subcore_barrier()

    pltpu.sync_copy(dst_hbm.at[1, wid], dst_v)
    ones = ones_v

    def group(g, c):
        for i in range(DFL):
            pltpu.async_copy(ones, acc.at[dst_v.at[g * DFL + i]],
                             dsem, add=True)
        for i in range(DFL):
            pltpu.make_async_copy(
                ones, acc.at[dst_v.at[g * DFL + i]], dsem).wait()
        return c
    lax.fori_loop(0, KN // DFL, group, 0)
    plsc.subcore_barrier()

    @pl.when(sid < 10)
    def _out():
        pltpu.sync_copy(acc.at[pl.ds(sid * 1000, 1000)],
                        obuf.at[pl.ds(0, 1000)])
        pltpu.sync_copy(obuf.at[pl.ds(0, 1000)],
                        out_hbm.at[pl.ds(cid * N + sid * 1000, 1000)])


_deg = functools.partial(
    pl.kernel,
    out_type=jax.ShapeDtypeStruct((NC * N,), jnp.float32),
    mesh=_mesh(),
    compiler_params=_SC_PARAMS,
    scratch_types=[
        pltpu.VMEM_SHARED((N + NDUMP,), jnp.float32),
        pltpu.VMEM((KN, KB), jnp.int32),
        pltpu.VMEM((KB,), jnp.float32),
        pltpu.VMEM((1008,), jnp.float32),
        pltpu.SemaphoreType.DMA,
    ])(_deg_body)


def _make_agg_body(NBUF, H):
    NG = KN // NBUF

    def _agg_body(h_hbm, e_hbm, out_hbm, acc, src_v, dst_v, *ring):
        rows = ring[:NBUF]
        gsem = ring[NBUF:2 * NBUF]
        ssem = ring[2 * NBUF:3 * NBUF]
        cid = lax.axis_index("c")
        sid = lax.axis_index("s")
        wid = sid * NC + cid
        d = rows[0].shape[1]

        def zfill(i, c):
            for j in range(d // 16):
                rows[0][i, pl.ds(j * 16, 16)] = jnp.zeros((16,), jnp.float32)
            return c
        lax.fori_loop(0, KB, zfill, 0)
        r0 = sid * RPT
        for ch in IOCH:
            pltpu.sync_copy(rows[0].at[pl.ds(0, ch)], acc.at[pl.ds(r0, ch)])
            r0 = r0 + ch
        plsc.subcore_barrier()

        pltpu.sync_copy(e_hbm.at[0, wid], src_v)
        pltpu.sync_copy(e_hbm.at[1, wid], dst_v)

        for b in range(H):
            pltpu.async_copy(h_hbm.at[src_v.at[b]], rows[b], gsem[b])

        def group(g, c):
            for b in range(NBUF):
                j = g * NBUF + b
                pltpu.make_async_copy(
                    h_hbm.at[src_v.at[j]], rows[b], gsem[b]).wait()
                pltpu.async_copy(rows[b], acc.at[dst_v.at[j]],
                                 ssem[b], add=True)
                h = (b + H) % NBUF
                jg = j + H

                @pl.when(jg < KN)
                def _prefetch(h=h, jg=jg):
                    @pl.when(jg >= NBUF)
                    def _wait_scatter():
                        pltpu.make_async_copy(
                            rows[h], acc.at[dst_v.at[jg - NBUF]],
                            ssem[h]).wait()
                    pltpu.async_copy(h_hbm.at[src_v.at[jg]], rows[h], gsem[h])
            return c
        lax.fori_loop(0, NG, group, 0)
        for b in range(NBUF):
            pltpu.make_async_copy(
                rows[b], acc.at[dst_v.at[KN - NBUF + b]], ssem[b]).wait()
        plsc.subcore_barrier()

        r1 = sid * RPT
        for k, ch in enumerate(IOCH):
            pltpu.sync_copy(acc.at[pl.ds(r1, ch)],
                            rows[k % NBUF].at[pl.ds(0, ch)])
            pltpu.sync_copy(rows[k % NBUF].at[pl.ds(0, ch)],
                            out_hbm.at[cid, pl.ds(r1, ch)])
            r1 = r1 + ch

    return _agg_body


def _make_agg(d, nbuf, h):
    return functools.partial(
        pl.kernel,
        out_type=jax.ShapeDtypeStruct((NC, N, d), jnp.float32),
        mesh=_mesh(),
        compiler_params=_SC_PARAMS,
        scratch_types=(
            [pltpu.VMEM_SHARED((N + NDUMP, d), jnp.float32),
             pltpu.VMEM((KN, KB), jnp.int32),
             pltpu.VMEM((KN, KB), jnp.int32)]
            + [pltpu.VMEM((KB, d), jnp.float32) for _ in range(nbuf)]
            + [pltpu.SemaphoreType.DMA for _ in range(2 * nbuf)]
        ))(_make_agg_body(nbuf, h))


_agg64 = _make_agg(D_HID, 8, 7)
_agg16 = _make_agg(D_OUT, 8, 7)


BR = 5000


def _tc1_body(x_ref, w_ref, deg_ref, h_ref, dinv_ref):
    dinv = lax.rsqrt(deg_ref[...] + 1.0)
    h = jnp.dot(x_ref[...], w_ref[...], preferred_element_type=jnp.float32)
    h_ref[...] = h * dinv
    dinv_ref[...] = dinv


_tc1 = pl.pallas_call(
    _tc1_body,
    grid=(N // BR,),
    in_specs=[pl.BlockSpec((BR, D_IN), lambda i: (i, 0)),
              pl.BlockSpec((D_IN, D_HID), lambda i: (0, 0)),
              pl.BlockSpec((BR, 1), lambda i: (i, 0))],
    out_specs=[pl.BlockSpec((BR, D_HID), lambda i: (i, 0)),
               pl.BlockSpec((BR, 1), lambda i: (i, 0))],
    out_shape=[jax.ShapeDtypeStruct((N, D_HID), jnp.float32),
               jax.ShapeDtypeStruct((N, 1), jnp.float32)])


def _tc_mid_body(p_ref, hp_ref, dinv_ref, b_ref, w_ref, out_ref):
    t = p_ref[0] + p_ref[1] + hp_ref[...]
    t = t * dinv_ref[...] + b_ref[...]
    t = jnp.maximum(t, 0.0)
    out_ref[...] = jnp.dot(
        t, w_ref[...], preferred_element_type=jnp.float32) * dinv_ref[...]


_tc_mid = pl.pallas_call(
    _tc_mid_body,
    grid=(N // BR,),
    in_specs=[pl.BlockSpec((NC, BR, D_HID), lambda i: (0, i, 0)),
              pl.BlockSpec((BR, D_HID), lambda i: (i, 0)),
              pl.BlockSpec((BR, 1), lambda i: (i, 0)),
              pl.BlockSpec((1, D_HID), lambda i: (0, 0)),
              pl.BlockSpec((D_HID, D_OUT), lambda i: (0, 0))],
    out_specs=pl.BlockSpec((BR, D_OUT), lambda i: (i, 0)),
    out_shape=jax.ShapeDtypeStruct((N, D_OUT), jnp.float32))


def _tc_out_body(p_ref, hp_ref, dinv_ref, b_ref, out_ref):
    t = (p_ref[0] + p_ref[1] + hp_ref[...]) * dinv_ref[...] + b_ref[...]
    m = jnp.max(t, axis=1, keepdims=True)
    e = jnp.exp(t - m)
    s = jnp.sum(e, axis=1, keepdims=True)
    out_ref[...] = (t - m) - jnp.log(s)


_tc_out = pl.pallas_call(
    _tc_out_body,
    grid=(N // BR,),
    in_specs=[pl.BlockSpec((NC, BR, D_OUT), lambda i: (0, i, 0)),
              pl.BlockSpec((BR, D_OUT), lambda i: (i, 0)),
              pl.BlockSpec((BR, 1), lambda i: (i, 0)),
              pl.BlockSpec((1, D_OUT), lambda i: (0, 0))],
    out_specs=pl.BlockSpec((BR, D_OUT), lambda i: (i, 0)),
    out_shape=jax.ShapeDtypeStruct((N, D_OUT), jnp.float32))


def kernel(x, edge_index, W1, b1, W2, b2):
    ei = edge_index.astype(jnp.int32)
    pad = jnp.arange(EPAD, dtype=jnp.int32)
    epad = jnp.stack([(pad * 8) % N, N + (pad % NDUMP)])
    eslab = jnp.concatenate([ei, epad], axis=1).reshape(2, NW, KN, KB)
    degp = _deg(eslab).reshape(NC, N)
    deg = (degp[0] + degp[1]).reshape(N, 1)
    h1p, dinv = _tc1(x, W1, deg)
    p1 = _agg64(h1p, eslab)
    h2p = _tc_mid(p1, h1p, dinv, b1.reshape(1, D_HID), W2)
    p2 = _agg16(h2p, eslab)
    return _tc_out(p2, h2p, dinv, b2.reshape(1, D_OUT))
